# baseline (speedup 1.0000x reference)
import functools

import jax
import jax.numpy as jnp
from jax import lax
from jax.experimental import pallas as pl
from jax.experimental.pallas import tpu as pltpu
from jax.experimental.pallas import tpu_sc as plsc

C, T, H, W = 3, 64, 384, 384
TS = T // 4
SLOW_T = [(i * 21) // 5 for i in range(TS)]
PPF = 6
QROWS = H // PPF
NW = 32
PER_W = C * TS * PPF // NW
NBUF = 5


def _sc_slow_gather(frames):
    mesh = plsc.VectorSubcoreMesh(core_axis_name="c", subcore_axis_name="s")

    @functools.partial(
        pl.kernel,
        mesh=mesh,
        out_type=jax.ShapeDtypeStruct((C, TS, H, W), jnp.float32),
        scratch_types=[
            pltpu.VMEM((NBUF, QROWS, W), jnp.float32),
            pltpu.SemaphoreType.DMA((NBUF,)),
            pltpu.SemaphoreType.DMA((NBUF,)),
        ],
        compiler_params=pltpu.CompilerParams(use_tc_tiling_on_sc=True),
    )
    def k(src, slow_out, buf, sem_r, sem_w):
        wid = lax.axis_index("s") * 2 + lax.axis_index("c")

        def coords(j):
            p = wid * PER_W + j
            c = p // (TS * PPF)
            i = (p // PPF) % TS
            q = p % PPF
            return c, i, q

        def rd(j):
            c, i, q = coords(j)
            rows = pl.ds(q * QROWS, QROWS)
            t = (i * 21) // 5
            return pltpu.make_async_copy(
                src.at[c, t, rows],
                buf.at[j % NBUF], sem_r.at[j % NBUF],
            )

        def wr(j):
            c, i, q = coords(j)
            rows = pl.ds(q * QROWS, QROWS)
            return pltpu.make_async_copy(
                buf.at[j % NBUF], slow_out.at[c, i, rows], sem_w.at[j % NBUF]
            )

        rd(0).start()
        rd(1).start()
        for j in range(PER_W):
            rd(j).wait()
            if j >= 3:
                wr(j - 3).wait()
            wr(j).start()
            if j + 2 < PER_W:
                rd(j + 2).start()
        for j in range(PER_W - 3, PER_W):
            wr(j).wait()

    return k(frames)


_CHUNK = 4
_RUNS = [(c, s, _CHUNK) for c in range(C) for s in range(0, T, _CHUNK)]


def _tc_fast_copy(frames):
    runs = _RUNS
    n = len(runs)
    NB = 6
    LOOK = 4

    def body(src_ref, out_ref, buf, sem_r, sem_w):
        def rd(k):
            c, s, ln = runs[k]
            return pltpu.make_async_copy(
                src_ref.at[c, pl.ds(s, ln)],
                buf.at[k % NB],
                sem_r.at[k % NB],
            )

        def wr(k):
            c, s, ln = runs[k]
            return pltpu.make_async_copy(
                buf.at[k % NB],
                out_ref.at[c, pl.ds(s, ln)],
                sem_w.at[k % NB],
            )

        for k in range(LOOK):
            rd(k).start()
        for k in range(n):
            rd(k).wait()
            wr(k).start()
            if k + LOOK < n:
                if k + LOOK >= NB:
                    wr(k + LOOK - NB).wait()
                rd(k + LOOK).start()
        for k in range(n - NB, n):
            wr(k).wait()

    return pl.pallas_call(
        body,
        in_specs=[pl.BlockSpec(memory_space=pl.ANY)],
        out_specs=pl.BlockSpec(memory_space=pl.ANY),
        out_shape=jax.ShapeDtypeStruct((C, T, H, W), jnp.float32),
        scratch_shapes=[
            pltpu.VMEM((NB, _CHUNK, H, W), jnp.float32),
            pltpu.SemaphoreType.DMA((NB,)),
            pltpu.SemaphoreType.DMA((NB,)),
        ],
    )(frames)


def kernel(frames):
    slow = _sc_slow_gather(frames)
    fast = _tc_fast_copy(frames)
    return (slow, fast)


# speedup vs baseline: 1.0249x; 1.0249x over previous
import functools

import jax
import jax.numpy as jnp
from jax import lax
from jax.experimental import pallas as pl
from jax.experimental.pallas import tpu as pltpu
from jax.experimental.pallas import tpu_sc as plsc

C, T, H, W = 3, 64, 384, 384
TS = T // 4
SLOW_T = [(i * 21) // 5 for i in range(TS)]
PPF = 6
QROWS = H // PPF
NW = 32
PER_W = C * TS * PPF // NW
NBUF = 5


def _sc_slow_gather(frames):
    mesh = plsc.VectorSubcoreMesh(core_axis_name="c", subcore_axis_name="s")

    @functools.partial(
        pl.kernel,
        mesh=mesh,
        out_type=(
            jax.ShapeDtypeStruct((C, TS, H, W), jnp.float32),
            jax.ShapeDtypeStruct((C, T, H, W), jnp.float32),
        ),
        scratch_types=[
            pltpu.VMEM((NBUF, QROWS, W), jnp.float32),
            pltpu.SemaphoreType.DMA((NBUF,)),
            pltpu.SemaphoreType.DMA((NBUF,)),
            pltpu.SemaphoreType.DMA((NBUF,)),
        ],
        compiler_params=pltpu.CompilerParams(use_tc_tiling_on_sc=True),
    )
    def k(src, slow_out, fastp_out, buf, sem_r, sem_ws, sem_wf):
        wid = lax.axis_index("s") * 2 + lax.axis_index("c")

        def coords(j):
            p = wid * PER_W + j
            c = p // (TS * PPF)
            i = (p // PPF) % TS
            q = p % PPF
            return c, i, q

        def rd(j):
            c, i, q = coords(j)
            rows = pl.ds(q * QROWS, QROWS)
            t = (i * 21) // 5
            return pltpu.make_async_copy(
                src.at[c, t, rows],
                buf.at[j % NBUF], sem_r.at[j % NBUF],
            )

        def wrs(j):
            c, i, q = coords(j)
            rows = pl.ds(q * QROWS, QROWS)
            return pltpu.make_async_copy(
                buf.at[j % NBUF], slow_out.at[c, i, rows], sem_ws.at[j % NBUF]
            )

        def wrf(j):
            c, i, q = coords(j)
            rows = pl.ds(q * QROWS, QROWS)
            t = (i * 21) // 5
            return pltpu.make_async_copy(
                buf.at[j % NBUF],
                fastp_out.at[c, t, rows],
                sem_wf.at[j % NBUF],
            )

        rd(0).start()
        rd(1).start()
        for j in range(PER_W):
            rd(j).wait()
            if j >= 3:
                wrs(j - 3).wait()
                wrf(j - 3).wait()
            wrs(j).start()
            wrf(j).start()
            if j + 2 < PER_W:
                rd(j + 2).start()
        for j in range(PER_W - 3, PER_W):
            wrs(j).wait()
            wrf(j).wait()

    return k(frames)


def _runs_nonslow():
    slow = set(SLOW_T)
    runs = []
    for c in range(C):
        t = 0
        while t < T:
            if t in slow:
                t += 1
                continue
            start = t
            while t < T and t not in slow:
                t += 1
            runs.append((c, start, t - start))
    return runs


def _tc_fast_fill(frames, fastp):
    runs = _runs_nonslow()
    n = len(runs)
    maxlen = max(r[2] for r in runs)
    NB = 8
    LOOK = 6

    def body(src_ref, part_ref, out_ref, buf, sem_r, sem_w):
        def rd(k):
            c, s, ln = runs[k]
            return pltpu.make_async_copy(
                src_ref.at[c, pl.ds(s, ln)],
                buf.at[k % NB, pl.ds(0, ln)],
                sem_r.at[k % NB],
            )

        def wr(k):
            c, s, ln = runs[k]
            return pltpu.make_async_copy(
                buf.at[k % NB, pl.ds(0, ln)],
                out_ref.at[c, pl.ds(s, ln)],
                sem_w.at[k % NB],
            )

        for k in range(LOOK):
            rd(k).start()
        for k in range(n):
            rd(k).wait()
            wr(k).start()
            if k + LOOK < n:
                if k + LOOK >= NB:
                    wr(k + LOOK - NB).wait()
                rd(k + LOOK).start()
        for k in range(n - NB, n):
            wr(k).wait()

    return pl.pallas_call(
        body,
        in_specs=[
            pl.BlockSpec(memory_space=pl.ANY),
            pl.BlockSpec(memory_space=pl.ANY),
        ],
        out_specs=pl.BlockSpec(memory_space=pl.ANY),
        out_shape=jax.ShapeDtypeStruct((C, T, H, W), jnp.float32),
        input_output_aliases={1: 0},
        scratch_shapes=[
            pltpu.VMEM((NB, maxlen, H, W), jnp.float32),
            pltpu.SemaphoreType.DMA((NB,)),
            pltpu.SemaphoreType.DMA((NB,)),
        ],
    )(frames, fastp)


def kernel(frames):
    slow, fastp = _sc_slow_gather(frames)
    fast = _tc_fast_fill(frames, fastp)
    return (slow, fast)
